# baseline (speedup 1.0000x reference)
import jax
import jax.numpy as jnp
from jax.experimental import pallas as pl
from jax.experimental.pallas import tpu as pltpu

IN_DIM = 512
MID = 1024
H, W = 50, 100
NPIX = H * W
MT = 1000
GRID = 5
GPAD = 104
G_ROWS = 5208
SLICE_ROWS = 1208
NOUT = 64

_OFF = lambda kh, kw: kh * W + kw + 3


def _rpn_kernel(g_ref, w9_hbm, wcr_ref, brpn_ref, bcr_ref,
                cls_ref, reg_ref, w9v_ref, sem):
    i = pl.program_id(0)

    def _tap_copy(t):
        return pltpu.make_async_copy(
            w9_hbm.at[pl.ds(t * IN_DIM, IN_DIM), :],
            w9v_ref.at[pl.ds(t * IN_DIM, IN_DIM), :],
            sem.at[t])

    @pl.when(i == 0)
    def _fetch_w9():
        for t in range(9):
            _tap_copy(t).start()
        for t in range(9):
            _tap_copy(t).wait()

    base = i * MT
    g = g_ref[pl.ds(base, SLICE_ROWS), :]
    j = jax.lax.broadcasted_iota(jnp.int32, (MT, IN_DIM), 0) % W
    acc = jnp.zeros((MT, MID), dtype=jnp.float32)
    for kh in range(3):
        for kw in range(3):
            lhs = jax.lax.slice_in_dim(g, _OFF(kh, kw), _OFF(kh, kw) + MT,
                                       axis=0)
            if kw == 0:
                lhs = jnp.where(j == 0, jnp.bfloat16(0), lhs)
            elif kw == 2:
                lhs = jnp.where(j == W - 1, jnp.bfloat16(0), lhs)
            t = kh * 3 + kw
            rhs = w9v_ref[t * IN_DIM:(t + 1) * IN_DIM, :]
            acc = acc + jnp.dot(lhs, rhs, preferred_element_type=jnp.float32)
    h = (jnp.maximum(acc + brpn_ref[0, :][None, :], 0.0)).astype(jnp.bfloat16)
    out2 = jnp.dot(h, wcr_ref[...],
                   preferred_element_type=jnp.float32) + bcr_ref[0, :][None, :]

    col = jax.lax.broadcasted_iota(jnp.int32, (MT, NOUT), 1)
    partner = jnp.where(col % 2 == 0, jnp.roll(out2, -1, axis=1),
                        jnp.roll(out2, 1, axis=1))
    m = jnp.maximum(out2, partner)
    e = jnp.exp(out2 - m)
    soft = e / (e + jnp.exp(partner - m))
    cls_ref[...] = jax.lax.slice_in_dim(soft, 0, 18, axis=1)
    reg_ref[...] = jax.lax.slice_in_dim(out2, 18, 54, axis=1)


def kernel(x, W_rpn, b_rpn, W_cls, b_cls, W_reg, b_reg):
    xt = jnp.transpose(x[0], (1, 2, 0)).reshape(NPIX, IN_DIM)
    g = jnp.pad(xt, ((GPAD, G_ROWS - GPAD - NPIX), (0, 0)))
    g = g.astype(jnp.bfloat16)

    w9 = jnp.transpose(W_rpn, (2, 3, 1, 0)).reshape(9 * IN_DIM, MID)
    w9 = w9.astype(jnp.bfloat16)
    wcr = jnp.concatenate([W_cls[:, :, 0, 0], W_reg[:, :, 0, 0]], axis=0)
    wcr = jnp.pad(wcr, ((0, NOUT - 54), (0, 0))).T.astype(jnp.bfloat16)
    bcr = jnp.pad(jnp.concatenate([b_cls, b_reg]), (0, NOUT - 54))

    cls_out, reg_out = pl.pallas_call(
        _rpn_kernel,
        grid=(GRID,),
        in_specs=[
            pl.BlockSpec((G_ROWS, IN_DIM), lambda i: (0, 0)),
            pl.BlockSpec(memory_space=pl.ANY),
            pl.BlockSpec((MID, NOUT), lambda i: (0, 0)),
            pl.BlockSpec((1, MID), lambda i: (0, 0)),
            pl.BlockSpec((1, NOUT), lambda i: (0, 0)),
        ],
        out_specs=[pl.BlockSpec((MT, 18), lambda i: (i, 0)),
                   pl.BlockSpec((MT, 36), lambda i: (i, 0))],
        out_shape=[jax.ShapeDtypeStruct((NPIX, 18), jnp.float32),
                   jax.ShapeDtypeStruct((NPIX, 36), jnp.float32)],
        scratch_shapes=[pltpu.VMEM((9 * IN_DIM, MID), jnp.bfloat16),
                        pltpu.SemaphoreType.DMA((9,))],
        compiler_params=pltpu.CompilerParams(
            dimension_semantics=("arbitrary",),
        ),
    )(g, w9, wcr, b_rpn[None, :], bcr[None, :])

    return (cls_out.reshape(NPIX * 9, 2), reg_out.reshape(NPIX * 9, 4))
